# Initial kernel scaffold; baseline (speedup 1.0000x reference)
import jax
import jax.numpy as jnp
from jax.experimental import pallas as pl


def kernel(x_a, x_b, edge_index_ab, edge_index_ba, W_ab_0, W_ba_0, W_self_a_0, W_self_b_0, b_a_0, b_b_0, W_ab_1, W_ba_1, W_self_a_1, W_self_b_1, b_a_1, b_b_1, W_h, b_h, W_o, b_o):
    raise NotImplementedError("write your pallas kernel here")


# speedup vs baseline: 2.6790x; 2.6790x over previous
import functools

import jax
import jax.numpy as jnp
from jax import lax
from jax.experimental import pallas as pl
from jax.experimental.pallas import tpu as pltpu
from jax.experimental.pallas import tpu_sc as plsc

_N = 10000
_D = 128
_OUT = 64
_E = 320000
_NR = 10112
_NS = 16
_CH = 128
_CHB = 16
_NBLK = 10
_NCH = _CHB * _NBLK
_EP = _NS * _NCH * _CH
_RPT = _NR // _NS


def _segsum_kernel(with_counts):
    def body(tab0, tab1, src_idx, dst_idx, z128,
             *out_and_scratch):
        if with_counts:
            (s_out, c_out, acc, src_c, dst_c, rows_v, cnt_loc,
             sem) = out_and_scratch
        else:
            (s_out, acc, src_c, dst_c, rows_v, sem) = out_and_scratch
        cid = lax.axis_index("c")
        sid = lax.axis_index("s")
        r0 = sid * _RPT
        row_chunks = [(0, 128), (128, 128), (256, 128), (384, 128),
                      (512, _RPT - 512)]

        pltpu.sync_copy(z128, rows_v)
        for off, n in row_chunks:
            pltpu.sync_copy(rows_v.at[pl.ds(0, n)],
                            acc.at[pl.ds(r0 + off, n)])
        if with_counts:
            z16v = jnp.zeros((16,), jnp.float32)

            def zstep(i, carry):
                cnt_loc[pl.ds(i * 16, 16)] = z16v
                return carry
            lax.fori_loop(0, 10240 // 16, zstep, 0)
        plsc.subcore_barrier()

        one16 = jnp.ones((16,), jnp.float32)

        def run(tab):
            def step(j, carry):
                pltpu.sync_copy(src_idx.at[cid, sid, j], src_c)
                pltpu.sync_copy(dst_idx.at[cid, sid, j], dst_c)
                pltpu.async_copy(tab.at[src_c], rows_v, sem).wait()
                pltpu.sync_copy(rows_v, acc.at[dst_c], add=True)
                if with_counts:
                    for k in range(_CH // 16):
                        d = dst_c[pl.ds(k * 16, 16)]
                        plsc.addupdate_scatter(cnt_loc, [d], one16)
                return carry
            lax.fori_loop(0, _NCH, step, 0)

        pl.when(cid == 0)(lambda: run(tab0))
        pl.when(cid == 1)(lambda: run(tab1))
        if with_counts:
            pltpu.sync_copy(cnt_loc, c_out.at[cid, sid])
        plsc.subcore_barrier()

        for off, n in row_chunks:
            pltpu.sync_copy(acc.at[pl.ds(r0 + off, n)],
                            rows_v.at[pl.ds(0, n)])
            pltpu.sync_copy(rows_v.at[pl.ds(0, n)],
                            s_out.at[cid, pl.ds(r0 + off, n)])

    out_type = [jax.ShapeDtypeStruct((2, _NR, _D), jnp.float32)]
    scratch = [pltpu.VMEM_SHARED((_NR, _D), jnp.float32)]
    if with_counts:
        out_type.append(jax.ShapeDtypeStruct((2, _NS, 10240), jnp.float32))
    scratch += [
        pltpu.VMEM((_CH,), jnp.int32),
        pltpu.VMEM((_CH,), jnp.int32),
        pltpu.VMEM((_CH, _D), jnp.float32),
    ]
    if with_counts:
        scratch.append(pltpu.VMEM((10240,), jnp.float32))
    scratch.append(pltpu.SemaphoreType.DMA)

    mesh = plsc.VectorSubcoreMesh(core_axis_name="c", subcore_axis_name="s")
    return pl.kernel(body, out_type=tuple(out_type), mesh=mesh,
                     scratch_types=tuple(scratch),
                     compiler_params=pltpu.CompilerParams(
                         needs_layout_passes=False))


def _cnt_reduce_body(c_in, c_out):
    c_out[...] = jnp.sum(c_in[...], axis=1)


def _tc_cnt_reduce(craw):
    return pl.pallas_call(
        _cnt_reduce_body,
        grid=(1,),
        in_specs=[pl.BlockSpec((2, _NS, 10240), lambda i: (0, 0, 0))],
        out_specs=pl.BlockSpec((2, 10240), lambda i: (0, 0)),
        out_shape=jax.ShapeDtypeStruct((2, 10240), jnp.float32),
    )(craw)


def _layer_body(xa, xb, sa, sb, ca, cb, wab, wba, wsa, wsb, ba, bb,
                a_out, b_out):
    agg_a = sa[...] / jnp.maximum(ca[...], 1.0)
    agg_b = sb[...] / jnp.maximum(cb[...], 1.0)
    f32 = jnp.float32
    na = (jnp.dot(xa[...], wsa[...], preferred_element_type=f32)
          + jnp.dot(agg_a, wba[...], preferred_element_type=f32) + ba[...])
    nb = (jnp.dot(xb[...], wsb[...], preferred_element_type=f32)
          + jnp.dot(agg_b, wab[...], preferred_element_type=f32) + bb[...])
    a_out[...] = jnp.maximum(na, 0.0)
    b_out[...] = jnp.maximum(nb, 0.0)


_BLK = 1000
_GRID = _N // _BLK


def _node_specs():
    rows = pl.BlockSpec((_BLK, _D), lambda i: (i, 0))
    cnt = pl.BlockSpec((_BLK, 1), lambda i: (i, 0))
    full = pl.BlockSpec((_D, _D), lambda i: (0, 0))
    bias = pl.BlockSpec((1, _D), lambda i: (0, 0))
    return rows, cnt, full, bias


def _tc_layer(xa, xb, sa, sb, ca, cb, wab, wba, wsa, wsb, ba, bb):
    rows, cnt, full, bias = _node_specs()
    return pl.pallas_call(
        _layer_body,
        grid=(_GRID,),
        in_specs=[rows, rows, rows, rows, cnt, cnt,
                  full, full, full, full, bias, bias],
        out_specs=[rows, rows],
        out_shape=[jax.ShapeDtypeStruct((_N, _D), jnp.float32)] * 2,
    )(xa, xb, sa, sb, ca, cb, wab, wba, wsa, wsb, ba, bb)


def _layer2_body(xa, xb, sa, sb, ca, cb, wab, wba, wsa, wsb, ba, bb,
                 wh, bh, wo, bo, out, pa, pb):
    i = pl.program_id(0)
    agg_a = sa[...] / jnp.maximum(ca[...], 1.0)
    agg_b = sb[...] / jnp.maximum(cb[...], 1.0)
    f32 = jnp.float32
    na = (jnp.dot(xa[...], wsa[...], preferred_element_type=f32)
          + jnp.dot(agg_a, wba[...], preferred_element_type=f32) + ba[...])
    nb = (jnp.dot(xb[...], wsb[...], preferred_element_type=f32)
          + jnp.dot(agg_b, wab[...], preferred_element_type=f32) + bb[...])
    na = jnp.maximum(na, 0.0)
    nb = jnp.maximum(nb, 0.0)

    @pl.when(i == 0)
    def _():
        pa[...] = jnp.zeros_like(pa)
        pb[...] = jnp.zeros_like(pb)

    pa[...] += jnp.sum(na, axis=0, keepdims=True)
    pb[...] += jnp.sum(nb, axis=0, keepdims=True)

    @pl.when(i == pl.num_programs(0) - 1)
    def _():
        pooled = jnp.concatenate([pa[...], pb[...]], axis=1)
        h = jnp.maximum(
            jnp.dot(pooled, wh[...], preferred_element_type=f32) + bh[...],
            0.0)
        out[...] = jnp.dot(h, wo[...], preferred_element_type=f32) + bo[...]


def _tc_layer2(xa, xb, sa, sb, ca, cb, wab, wba, wsa, wsb, ba, bb,
               wh, bh, wo, bo):
    rows, cnt, full, bias = _node_specs()
    return pl.pallas_call(
        _layer2_body,
        grid=(_GRID,),
        in_specs=[rows, rows, rows, rows, cnt, cnt,
                  full, full, full, full, bias, bias,
                  pl.BlockSpec((2 * _D, _D), lambda i: (0, 0)),
                  bias,
                  pl.BlockSpec((_D, _OUT), lambda i: (0, 0)),
                  pl.BlockSpec((1, _OUT), lambda i: (0, 0))],
        out_specs=pl.BlockSpec((1, _OUT), lambda i: (0, 0)),
        out_shape=jax.ShapeDtypeStruct((1, _OUT), jnp.float32),
        scratch_shapes=[pltpu.VMEM((1, _D), jnp.float32),
                        pltpu.VMEM((1, _D), jnp.float32)],
    )(xa, xb, sa, sb, ca, cb, wab, wba, wsa, wsb, ba, bb, wh, bh, wo, bo)


def _prep_idx(ei):
    src, dst = ei[0], ei[1]
    pad = _EP - _E
    src = jnp.concatenate([src, jnp.zeros((pad,), jnp.int32)])
    dst = jnp.concatenate([dst, jnp.full((pad,), _N, jnp.int32)])
    return (src.reshape(_NS, _NCH, _CH), dst.reshape(_NS, _NCH, _CH))


def kernel(x_a, x_b, edge_index_ab, edge_index_ba, W_ab_0, W_ba_0,
           W_self_a_0, W_self_b_0, b_a_0, b_b_0, W_ab_1, W_ba_1,
           W_self_a_1, W_self_b_1, b_a_1, b_b_1, W_h, b_h, W_o, b_o):
    src_ab, dst_ab = _prep_idx(edge_index_ab)
    src_ba, dst_ba = _prep_idx(edge_index_ba)
    src_idx = jnp.stack([src_ab, src_ba])
    dst_idx = jnp.stack([dst_ab, dst_ba])

    z128 = jnp.zeros((_CH, _D), jnp.float32)

    seg0 = _segsum_kernel(True)
    seg1 = _segsum_kernel(False)

    s0, craw = seg0(x_a, x_b, src_idx, dst_idx, z128)
    cflat = _tc_cnt_reduce(craw)
    c_b = cflat[0, :_N].reshape(_N, 1)
    c_a = cflat[1, :_N].reshape(_N, 1)
    s_b0 = s0[0, :_N]
    s_a0 = s0[1, :_N]

    ba0 = b_a_0.reshape(1, _D)
    bb0 = b_b_0.reshape(1, _D)
    a1, b1 = _tc_layer(x_a, x_b, s_a0, s_b0, c_a, c_b,
                       W_ab_0, W_ba_0, W_self_a_0, W_self_b_0, ba0, bb0)

    (s1,) = seg1(a1, b1, src_idx, dst_idx, z128)
    s_b1 = s1[0, :_N]
    s_a1 = s1[1, :_N]

    out = _tc_layer2(a1, b1, s_a1, s_b1, c_a, c_b,
                     W_ab_1, W_ba_1, W_self_a_1, W_self_b_1,
                     b_a_1.reshape(1, _D), b_b_1.reshape(1, _D),
                     W_h, b_h.reshape(1, _D), W_o, b_o.reshape(1, _OUT))
    return out.reshape(_OUT)


# speedup vs baseline: 3.5082x; 1.3095x over previous
import functools

import jax
import jax.numpy as jnp
from jax import lax
from jax.experimental import pallas as pl
from jax.experimental.pallas import tpu as pltpu
from jax.experimental.pallas import tpu_sc as plsc

_N = 10000
_D = 128
_OUT = 64
_E = 320000
_NR = 10112
_NS = 16
_CH = 128
_CHB = 16
_NBLK = 10
_NCH = _CHB * _NBLK
_EP = _NS * _NCH * _CH
_RPT = _NR // _NS


def _segsum_kernel(with_counts):
    def body(tab0, tab1, src_idx, dst_idx, z128,
             *out_and_scratch):
        if with_counts:
            (s_out, c_out, acc, src_blk, dst_blk, rows0, rows1, cnt_loc,
             sem0, sem1) = out_and_scratch
        else:
            (s_out, acc, src_blk, dst_blk, rows0, rows1,
             sem0, sem1) = out_and_scratch
        rows = (rows0, rows1)
        sems = (sem0, sem1)
        cid = lax.axis_index("c")
        sid = lax.axis_index("s")
        r0 = sid * _RPT
        row_chunks = [(0, 128), (128, 128), (256, 128), (384, 128),
                      (512, _RPT - 512)]

        pltpu.sync_copy(z128, rows0)
        for off, n in row_chunks:
            pltpu.sync_copy(rows0.at[pl.ds(0, n)],
                            acc.at[pl.ds(r0 + off, n)])
        if with_counts:
            z16v = jnp.zeros((16,), jnp.float32)

            def zstep(i, carry):
                cnt_loc[pl.ds(i * 16, 16)] = z16v
                return carry
            lax.fori_loop(0, 10240 // 16, zstep, 0)
        plsc.subcore_barrier()

        one16 = jnp.ones((16,), jnp.float32)

        def run(tab):
            def block(b, carry):
                pltpu.sync_copy(src_idx.at[cid, sid, b], src_blk)
                pltpu.sync_copy(dst_idx.at[cid, sid, b], dst_blk)
                descs = [None, None]
                descs[0] = pltpu.async_copy(tab.at[src_blk.at[0]],
                                            rows[0], sems[0])
                for j in range(_CHB):
                    descs[j % 2].wait()
                    if j + 1 < _CHB:
                        descs[(j + 1) % 2] = pltpu.async_copy(
                            tab.at[src_blk.at[j + 1]],
                            rows[(j + 1) % 2], sems[(j + 1) % 2])
                    pltpu.sync_copy(rows[j % 2], acc.at[dst_blk.at[j]],
                                    add=True)
                    if with_counts:
                        for k in range(_CH // 16):
                            d = dst_blk[j, pl.ds(k * 16, 16)]
                            plsc.addupdate_scatter(cnt_loc, [d], one16)
                return carry
            lax.fori_loop(0, _NBLK, block, 0)

        pl.when(cid == 0)(lambda: run(tab0))
        pl.when(cid == 1)(lambda: run(tab1))
        if with_counts:
            pltpu.sync_copy(cnt_loc, c_out.at[cid, sid])
        plsc.subcore_barrier()

        for off, n in row_chunks:
            pltpu.sync_copy(acc.at[pl.ds(r0 + off, n)],
                            rows0.at[pl.ds(0, n)])
            pltpu.sync_copy(rows0.at[pl.ds(0, n)],
                            s_out.at[cid, pl.ds(r0 + off, n)])

    out_type = [jax.ShapeDtypeStruct((2, _NR, _D), jnp.float32)]
    scratch = [pltpu.VMEM_SHARED((_NR, _D), jnp.float32)]
    if with_counts:
        out_type.append(jax.ShapeDtypeStruct((2, _NS, 10240), jnp.float32))
    scratch += [
        pltpu.VMEM((_CHB, _CH), jnp.int32),
        pltpu.VMEM((_CHB, _CH), jnp.int32),
        pltpu.VMEM((_CH, _D), jnp.float32),
        pltpu.VMEM((_CH, _D), jnp.float32),
    ]
    if with_counts:
        scratch.append(pltpu.VMEM((10240,), jnp.float32))
    scratch.append(pltpu.SemaphoreType.DMA)
    scratch.append(pltpu.SemaphoreType.DMA)

    mesh = plsc.VectorSubcoreMesh(core_axis_name="c", subcore_axis_name="s")
    return pl.kernel(body, out_type=tuple(out_type), mesh=mesh,
                     scratch_types=tuple(scratch),
                     compiler_params=pltpu.CompilerParams(
                         needs_layout_passes=False))


def _cnt_reduce_body(c_in, c_out):
    c_out[...] = jnp.sum(c_in[...], axis=1)


def _tc_cnt_reduce(craw):
    return pl.pallas_call(
        _cnt_reduce_body,
        grid=(1,),
        in_specs=[pl.BlockSpec((2, _NS, 10240), lambda i: (0, 0, 0))],
        out_specs=pl.BlockSpec((2, 10240), lambda i: (0, 0)),
        out_shape=jax.ShapeDtypeStruct((2, 10240), jnp.float32),
    )(craw)


def _layer_body(xa, xb, sa, sb, ca, cb, wab, wba, wsa, wsb, ba, bb,
                a_out, b_out):
    agg_a = sa[...] / jnp.maximum(ca[...], 1.0)
    agg_b = sb[...] / jnp.maximum(cb[...], 1.0)
    f32 = jnp.float32
    na = (jnp.dot(xa[...], wsa[...], preferred_element_type=f32)
          + jnp.dot(agg_a, wba[...], preferred_element_type=f32) + ba[...])
    nb = (jnp.dot(xb[...], wsb[...], preferred_element_type=f32)
          + jnp.dot(agg_b, wab[...], preferred_element_type=f32) + bb[...])
    a_out[...] = jnp.maximum(na, 0.0)
    b_out[...] = jnp.maximum(nb, 0.0)


_BLK = 1000
_GRID = _N // _BLK


def _node_specs():
    rows = pl.BlockSpec((_BLK, _D), lambda i: (i, 0))
    cnt = pl.BlockSpec((_BLK, 1), lambda i: (i, 0))
    full = pl.BlockSpec((_D, _D), lambda i: (0, 0))
    bias = pl.BlockSpec((1, _D), lambda i: (0, 0))
    return rows, cnt, full, bias


def _tc_layer(xa, xb, sa, sb, ca, cb, wab, wba, wsa, wsb, ba, bb):
    rows, cnt, full, bias = _node_specs()
    return pl.pallas_call(
        _layer_body,
        grid=(_GRID,),
        in_specs=[rows, rows, rows, rows, cnt, cnt,
                  full, full, full, full, bias, bias],
        out_specs=[rows, rows],
        out_shape=[jax.ShapeDtypeStruct((_N, _D), jnp.float32)] * 2,
    )(xa, xb, sa, sb, ca, cb, wab, wba, wsa, wsb, ba, bb)


def _layer2_body(xa, xb, sa, sb, ca, cb, wab, wba, wsa, wsb, ba, bb,
                 wh, bh, wo, bo, out, pa, pb):
    i = pl.program_id(0)
    agg_a = sa[...] / jnp.maximum(ca[...], 1.0)
    agg_b = sb[...] / jnp.maximum(cb[...], 1.0)
    f32 = jnp.float32
    na = (jnp.dot(xa[...], wsa[...], preferred_element_type=f32)
          + jnp.dot(agg_a, wba[...], preferred_element_type=f32) + ba[...])
    nb = (jnp.dot(xb[...], wsb[...], preferred_element_type=f32)
          + jnp.dot(agg_b, wab[...], preferred_element_type=f32) + bb[...])
    na = jnp.maximum(na, 0.0)
    nb = jnp.maximum(nb, 0.0)

    @pl.when(i == 0)
    def _():
        pa[...] = jnp.zeros_like(pa)
        pb[...] = jnp.zeros_like(pb)

    pa[...] += jnp.sum(na, axis=0, keepdims=True)
    pb[...] += jnp.sum(nb, axis=0, keepdims=True)

    @pl.when(i == pl.num_programs(0) - 1)
    def _():
        pooled = jnp.concatenate([pa[...], pb[...]], axis=1)
        h = jnp.maximum(
            jnp.dot(pooled, wh[...], preferred_element_type=f32) + bh[...],
            0.0)
        out[...] = jnp.dot(h, wo[...], preferred_element_type=f32) + bo[...]


def _tc_layer2(xa, xb, sa, sb, ca, cb, wab, wba, wsa, wsb, ba, bb,
               wh, bh, wo, bo):
    rows, cnt, full, bias = _node_specs()
    return pl.pallas_call(
        _layer2_body,
        grid=(_GRID,),
        in_specs=[rows, rows, rows, rows, cnt, cnt,
                  full, full, full, full, bias, bias,
                  pl.BlockSpec((2 * _D, _D), lambda i: (0, 0)),
                  bias,
                  pl.BlockSpec((_D, _OUT), lambda i: (0, 0)),
                  pl.BlockSpec((1, _OUT), lambda i: (0, 0))],
        out_specs=pl.BlockSpec((1, _OUT), lambda i: (0, 0)),
        out_shape=jax.ShapeDtypeStruct((1, _OUT), jnp.float32),
        scratch_shapes=[pltpu.VMEM((1, _D), jnp.float32),
                        pltpu.VMEM((1, _D), jnp.float32)],
    )(xa, xb, sa, sb, ca, cb, wab, wba, wsa, wsb, ba, bb, wh, bh, wo, bo)


def _prep_idx(ei):
    src, dst = ei[0], ei[1]
    pad = _EP - _E
    src = jnp.concatenate([src, jnp.zeros((pad,), jnp.int32)])
    dst = jnp.concatenate([dst, jnp.full((pad,), _N, jnp.int32)])
    return (src.reshape(_NS, _NBLK, _CHB, _CH),
            dst.reshape(_NS, _NBLK, _CHB, _CH))


def kernel(x_a, x_b, edge_index_ab, edge_index_ba, W_ab_0, W_ba_0,
           W_self_a_0, W_self_b_0, b_a_0, b_b_0, W_ab_1, W_ba_1,
           W_self_a_1, W_self_b_1, b_a_1, b_b_1, W_h, b_h, W_o, b_o):
    src_ab, dst_ab = _prep_idx(edge_index_ab)
    src_ba, dst_ba = _prep_idx(edge_index_ba)
    src_idx = jnp.stack([src_ab, src_ba])
    dst_idx = jnp.stack([dst_ab, dst_ba])

    z128 = jnp.zeros((_CH, _D), jnp.float32)

    seg0 = _segsum_kernel(True)
    seg1 = _segsum_kernel(False)

    s0, craw = seg0(x_a, x_b, src_idx, dst_idx, z128)
    cflat = _tc_cnt_reduce(craw)
    c_b = cflat[0, :_N].reshape(_N, 1)
    c_a = cflat[1, :_N].reshape(_N, 1)
    s_b0 = s0[0, :_N]
    s_a0 = s0[1, :_N]

    ba0 = b_a_0.reshape(1, _D)
    bb0 = b_b_0.reshape(1, _D)
    a1, b1 = _tc_layer(x_a, x_b, s_a0, s_b0, c_a, c_b,
                       W_ab_0, W_ba_0, W_self_a_0, W_self_b_0, ba0, bb0)

    (s1,) = seg1(a1, b1, src_idx, dst_idx, z128)
    s_b1 = s1[0, :_N]
    s_a1 = s1[1, :_N]

    out = _tc_layer2(a1, b1, s_a1, s_b1, c_a, c_b,
                     W_ab_1, W_ba_1, W_self_a_1, W_self_b_1,
                     b_a_1.reshape(1, _D), b_b_1.reshape(1, _D),
                     W_h, b_h.reshape(1, _D), W_o, b_o.reshape(1, _OUT))
    return out.reshape(_OUT)


# speedup vs baseline: 3.6995x; 1.0545x over previous
import functools

import jax
import jax.numpy as jnp
from jax import lax
from jax.experimental import pallas as pl
from jax.experimental.pallas import tpu as pltpu
from jax.experimental.pallas import tpu_sc as plsc

_N = 10000
_D = 128
_OUT = 64
_E = 320000
_NR = 10112
_NS = 16
_CH = 128
_CHB = 16
_NBLK = 10
_NCH = _CHB * _NBLK
_EP = _NS * _NCH * _CH
_RPT = _NR // _NS


def _segsum_kernel(with_counts):
    def body(tab0, tab1, src_idx, dst_idx, z128,
             *out_and_scratch):
        if with_counts:
            (s_out, c_out, acc, src_blk, dst_blk, rows0, rows1, cnt_loc,
             sem0, sem1, ssem0, ssem1) = out_and_scratch
        else:
            (s_out, acc, src_blk, dst_blk, rows0, rows1,
             sem0, sem1, ssem0, ssem1) = out_and_scratch
        rows = (rows0, rows1)
        sems = (sem0, sem1)
        ssems = (ssem0, ssem1)
        cid = lax.axis_index("c")
        sid = lax.axis_index("s")
        r0 = sid * _RPT
        row_chunks = [(0, 128), (128, 128), (256, 128), (384, 128),
                      (512, _RPT - 512)]

        pltpu.sync_copy(z128, rows0)
        for off, n in row_chunks:
            pltpu.sync_copy(rows0.at[pl.ds(0, n)],
                            acc.at[pl.ds(r0 + off, n)])
        if with_counts:
            z16v = jnp.zeros((16,), jnp.float32)

            def zstep(i, carry):
                cnt_loc[pl.ds(i * 16, 16)] = z16v
                return carry
            lax.fori_loop(0, 10240 // 16, zstep, 0)
        plsc.subcore_barrier()

        one16 = jnp.ones((16,), jnp.float32)

        def run(tab):
            def block(b, carry):
                pltpu.sync_copy(src_idx.at[cid, sid, b], src_blk)
                pltpu.sync_copy(dst_idx.at[cid, sid, b], dst_blk)
                gd = [None, None]
                sd = [None, None]
                gd[0] = pltpu.async_copy(tab.at[src_blk.at[0]],
                                         rows[0], sems[0])
                for j in range(_CHB):
                    p = j % 2
                    if j + 1 < _CHB:
                        if sd[1 - p] is not None:
                            sd[1 - p].wait()
                        gd[1 - p] = pltpu.async_copy(
                            tab.at[src_blk.at[j + 1]],
                            rows[1 - p], sems[1 - p])
                    gd[p].wait()
                    sd[p] = pltpu.async_copy(rows[p], acc.at[dst_blk.at[j]],
                                             ssems[p], add=True)
                    if with_counts:
                        for k in range(_CH // 16):
                            d = dst_blk[j, pl.ds(k * 16, 16)]
                            plsc.addupdate_scatter(cnt_loc, [d], one16)
                sd[0].wait()
                sd[1].wait()
                return carry
            lax.fori_loop(0, _NBLK, block, 0)

        pl.when(cid == 0)(lambda: run(tab0))
        pl.when(cid == 1)(lambda: run(tab1))
        if with_counts:
            pltpu.sync_copy(cnt_loc, c_out.at[cid, sid])
        plsc.subcore_barrier()

        for off, n in row_chunks:
            pltpu.sync_copy(acc.at[pl.ds(r0 + off, n)],
                            rows0.at[pl.ds(0, n)])
            pltpu.sync_copy(rows0.at[pl.ds(0, n)],
                            s_out.at[cid, pl.ds(r0 + off, n)])

    out_type = [jax.ShapeDtypeStruct((2, _NR, _D), jnp.float32)]
    scratch = [pltpu.VMEM_SHARED((_NR, _D), jnp.float32)]
    if with_counts:
        out_type.append(jax.ShapeDtypeStruct((2, _NS, 10240), jnp.float32))
    scratch += [
        pltpu.VMEM((_CHB, _CH), jnp.int32),
        pltpu.VMEM((_CHB, _CH), jnp.int32),
        pltpu.VMEM((_CH, _D), jnp.float32),
        pltpu.VMEM((_CH, _D), jnp.float32),
    ]
    if with_counts:
        scratch.append(pltpu.VMEM((10240,), jnp.float32))
    scratch += [pltpu.SemaphoreType.DMA] * 4

    mesh = plsc.VectorSubcoreMesh(core_axis_name="c", subcore_axis_name="s")
    return pl.kernel(body, out_type=tuple(out_type), mesh=mesh,
                     scratch_types=tuple(scratch),
                     compiler_params=pltpu.CompilerParams(
                         needs_layout_passes=False))


def _cnt_reduce_body(c_in, c_out):
    c_out[...] = jnp.sum(c_in[...], axis=1)


def _tc_cnt_reduce(craw):
    return pl.pallas_call(
        _cnt_reduce_body,
        grid=(1,),
        in_specs=[pl.BlockSpec((2, _NS, 10240), lambda i: (0, 0, 0))],
        out_specs=pl.BlockSpec((2, 10240), lambda i: (0, 0)),
        out_shape=jax.ShapeDtypeStruct((2, 10240), jnp.float32),
    )(craw)


def _layer_body(xa, xb, sa, sb, ca, cb, wab, wba, wsa, wsb, ba, bb,
                a_out, b_out):
    agg_a = sa[...] / jnp.maximum(ca[...], 1.0)
    agg_b = sb[...] / jnp.maximum(cb[...], 1.0)
    f32 = jnp.float32
    na = (jnp.dot(xa[...], wsa[...], preferred_element_type=f32)
          + jnp.dot(agg_a, wba[...], preferred_element_type=f32) + ba[...])
    nb = (jnp.dot(xb[...], wsb[...], preferred_element_type=f32)
          + jnp.dot(agg_b, wab[...], preferred_element_type=f32) + bb[...])
    a_out[...] = jnp.maximum(na, 0.0)
    b_out[...] = jnp.maximum(nb, 0.0)


_BLK = 1000
_GRID = _N // _BLK


def _node_specs():
    rows = pl.BlockSpec((_BLK, _D), lambda i: (i, 0))
    cnt = pl.BlockSpec((_BLK, 1), lambda i: (i, 0))
    full = pl.BlockSpec((_D, _D), lambda i: (0, 0))
    bias = pl.BlockSpec((1, _D), lambda i: (0, 0))
    return rows, cnt, full, bias


def _tc_layer(xa, xb, sa, sb, ca, cb, wab, wba, wsa, wsb, ba, bb):
    rows, cnt, full, bias = _node_specs()
    return pl.pallas_call(
        _layer_body,
        grid=(_GRID,),
        in_specs=[rows, rows, rows, rows, cnt, cnt,
                  full, full, full, full, bias, bias],
        out_specs=[rows, rows],
        out_shape=[jax.ShapeDtypeStruct((_N, _D), jnp.float32)] * 2,
    )(xa, xb, sa, sb, ca, cb, wab, wba, wsa, wsb, ba, bb)


def _layer2_body(xa, xb, sa, sb, ca, cb, wab, wba, wsa, wsb, ba, bb,
                 wh, bh, wo, bo, out, pa, pb):
    i = pl.program_id(0)
    agg_a = sa[...] / jnp.maximum(ca[...], 1.0)
    agg_b = sb[...] / jnp.maximum(cb[...], 1.0)
    f32 = jnp.float32
    na = (jnp.dot(xa[...], wsa[...], preferred_element_type=f32)
          + jnp.dot(agg_a, wba[...], preferred_element_type=f32) + ba[...])
    nb = (jnp.dot(xb[...], wsb[...], preferred_element_type=f32)
          + jnp.dot(agg_b, wab[...], preferred_element_type=f32) + bb[...])
    na = jnp.maximum(na, 0.0)
    nb = jnp.maximum(nb, 0.0)

    @pl.when(i == 0)
    def _():
        pa[...] = jnp.zeros_like(pa)
        pb[...] = jnp.zeros_like(pb)

    pa[...] += jnp.sum(na, axis=0, keepdims=True)
    pb[...] += jnp.sum(nb, axis=0, keepdims=True)

    @pl.when(i == pl.num_programs(0) - 1)
    def _():
        pooled = jnp.concatenate([pa[...], pb[...]], axis=1)
        h = jnp.maximum(
            jnp.dot(pooled, wh[...], preferred_element_type=f32) + bh[...],
            0.0)
        out[...] = jnp.dot(h, wo[...], preferred_element_type=f32) + bo[...]


def _tc_layer2(xa, xb, sa, sb, ca, cb, wab, wba, wsa, wsb, ba, bb,
               wh, bh, wo, bo):
    rows, cnt, full, bias = _node_specs()
    return pl.pallas_call(
        _layer2_body,
        grid=(_GRID,),
        in_specs=[rows, rows, rows, rows, cnt, cnt,
                  full, full, full, full, bias, bias,
                  pl.BlockSpec((2 * _D, _D), lambda i: (0, 0)),
                  bias,
                  pl.BlockSpec((_D, _OUT), lambda i: (0, 0)),
                  pl.BlockSpec((1, _OUT), lambda i: (0, 0))],
        out_specs=pl.BlockSpec((1, _OUT), lambda i: (0, 0)),
        out_shape=jax.ShapeDtypeStruct((1, _OUT), jnp.float32),
        scratch_shapes=[pltpu.VMEM((1, _D), jnp.float32),
                        pltpu.VMEM((1, _D), jnp.float32)],
    )(xa, xb, sa, sb, ca, cb, wab, wba, wsa, wsb, ba, bb, wh, bh, wo, bo)


def _prep_idx(ei):
    src, dst = ei[0], ei[1]
    pad = _EP - _E
    src = jnp.concatenate([src, jnp.zeros((pad,), jnp.int32)])
    dst = jnp.concatenate([dst, jnp.full((pad,), _N, jnp.int32)])
    return (src.reshape(_NS, _NBLK, _CHB, _CH),
            dst.reshape(_NS, _NBLK, _CHB, _CH))


def kernel(x_a, x_b, edge_index_ab, edge_index_ba, W_ab_0, W_ba_0,
           W_self_a_0, W_self_b_0, b_a_0, b_b_0, W_ab_1, W_ba_1,
           W_self_a_1, W_self_b_1, b_a_1, b_b_1, W_h, b_h, W_o, b_o):
    src_ab, dst_ab = _prep_idx(edge_index_ab)
    src_ba, dst_ba = _prep_idx(edge_index_ba)
    src_idx = jnp.stack([src_ab, src_ba])
    dst_idx = jnp.stack([dst_ab, dst_ba])

    z128 = jnp.zeros((_CH, _D), jnp.float32)

    seg0 = _segsum_kernel(True)
    seg1 = _segsum_kernel(False)

    s0, craw = seg0(x_a, x_b, src_idx, dst_idx, z128)
    cflat = _tc_cnt_reduce(craw)
    c_b = cflat[0, :_N].reshape(_N, 1)
    c_a = cflat[1, :_N].reshape(_N, 1)
    s_b0 = s0[0, :_N]
    s_a0 = s0[1, :_N]

    ba0 = b_a_0.reshape(1, _D)
    bb0 = b_b_0.reshape(1, _D)
    a1, b1 = _tc_layer(x_a, x_b, s_a0, s_b0, c_a, c_b,
                       W_ab_0, W_ba_0, W_self_a_0, W_self_b_0, ba0, bb0)

    (s1,) = seg1(a1, b1, src_idx, dst_idx, z128)
    s_b1 = s1[0, :_N]
    s_a1 = s1[1, :_N]

    out = _tc_layer2(a1, b1, s_a1, s_b1, c_a, c_b,
                     W_ab_1, W_ba_1, W_self_a_1, W_self_b_1,
                     b_a_1.reshape(1, _D), b_b_1.reshape(1, _D),
                     W_h, b_h.reshape(1, _D), W_o, b_o.reshape(1, _OUT))
    return out.reshape(_OUT)


# speedup vs baseline: 3.7260x; 1.0072x over previous
import functools

import jax
import jax.numpy as jnp
from jax import lax
from jax.experimental import pallas as pl
from jax.experimental.pallas import tpu as pltpu
from jax.experimental.pallas import tpu_sc as plsc

_N = 10000
_D = 128
_OUT = 64
_E = 320000
_NR = 10112
_NS = 16
_CH = 128
_CHB = 16
_NBLK = 10
_NCH = _CHB * _NBLK
_EP = _NS * _NCH * _CH
_RPT = _NR // _NS


def _segsum_kernel(with_counts):
    def body(tab0, tab1, edge_idx, z128,
             *out_and_scratch):
        if with_counts:
            (s_out, c_out, acc, idx_blk, rows0, rows1, cnt_loc,
             sem0, sem1, ssem0, ssem1) = out_and_scratch
        else:
            (s_out, acc, idx_blk, rows0, rows1,
             sem0, sem1, ssem0, ssem1) = out_and_scratch
        rows = (rows0, rows1)
        sems = (sem0, sem1)
        ssems = (ssem0, ssem1)
        cid = lax.axis_index("c")
        sid = lax.axis_index("s")
        r0 = sid * _RPT
        row_chunks = [(0, 128), (128, 128), (256, 128), (384, 128),
                      (512, _RPT - 512)]

        pltpu.sync_copy(z128, rows0)
        for off, n in row_chunks:
            pltpu.sync_copy(rows0.at[pl.ds(0, n)],
                            acc.at[pl.ds(r0 + off, n)])
        if with_counts:
            z16v = jnp.zeros((16,), jnp.float32)

            def zstep(i, carry):
                cnt_loc[pl.ds(i * 16, 16)] = z16v
                return carry
            lax.fori_loop(0, 10240 // 16, zstep, 0)
        plsc.subcore_barrier()

        one16 = jnp.ones((16,), jnp.float32)

        def run(tab):
            def block(b, carry):
                pltpu.sync_copy(edge_idx.at[cid, sid, b], idx_blk)
                gd = [None, None]
                sd = [None, None]
                gd[0] = pltpu.async_copy(tab.at[idx_blk.at[0, 0]],
                                         rows[0], sems[0])
                for j in range(_CHB):
                    p = j % 2
                    if j + 1 < _CHB:
                        if sd[1 - p] is not None:
                            sd[1 - p].wait()
                        gd[1 - p] = pltpu.async_copy(
                            tab.at[idx_blk.at[0, j + 1]],
                            rows[1 - p], sems[1 - p])
                    gd[p].wait()
                    sd[p] = pltpu.async_copy(rows[p],
                                             acc.at[idx_blk.at[1, j]],
                                             ssems[p], add=True)
                    if with_counts:
                        for k in range(_CH // 16):
                            d = idx_blk[1, j, pl.ds(k * 16, 16)]
                            plsc.addupdate_scatter(cnt_loc, [d], one16)
                sd[0].wait()
                sd[1].wait()
                return carry
            lax.fori_loop(0, _NBLK, block, 0)

        pl.when(cid == 0)(lambda: run(tab0))
        pl.when(cid == 1)(lambda: run(tab1))
        if with_counts:
            pltpu.sync_copy(cnt_loc, c_out.at[cid, sid])
        plsc.subcore_barrier()

        for off, n in row_chunks:
            pltpu.sync_copy(acc.at[pl.ds(r0 + off, n)],
                            rows0.at[pl.ds(0, n)])
            pltpu.sync_copy(rows0.at[pl.ds(0, n)],
                            s_out.at[cid, pl.ds(r0 + off, n)])

    out_type = [jax.ShapeDtypeStruct((2, _NR, _D), jnp.float32)]
    scratch = [pltpu.VMEM_SHARED((_NR, _D), jnp.float32)]
    if with_counts:
        out_type.append(jax.ShapeDtypeStruct((2, _NS, 10240), jnp.float32))
    scratch += [
        pltpu.VMEM((2, _CHB, _CH), jnp.int32),
        pltpu.VMEM((_CH, _D), jnp.float32),
        pltpu.VMEM((_CH, _D), jnp.float32),
    ]
    if with_counts:
        scratch.append(pltpu.VMEM((10240,), jnp.float32))
    scratch += [pltpu.SemaphoreType.DMA] * 4

    mesh = plsc.VectorSubcoreMesh(core_axis_name="c", subcore_axis_name="s")
    return pl.kernel(body, out_type=tuple(out_type), mesh=mesh,
                     scratch_types=tuple(scratch),
                     compiler_params=pltpu.CompilerParams(
                         needs_layout_passes=False))


def _cnt_reduce_body(c_in, c_out):
    c_out[...] = jnp.sum(c_in[...], axis=1)


def _tc_cnt_reduce(craw):
    return pl.pallas_call(
        _cnt_reduce_body,
        grid=(1,),
        in_specs=[pl.BlockSpec((2, _NS, 10240), lambda i: (0, 0, 0))],
        out_specs=pl.BlockSpec((2, 10240), lambda i: (0, 0)),
        out_shape=jax.ShapeDtypeStruct((2, 10240), jnp.float32),
    )(craw)


def _layer_body(xa, xb, sa, sb, ca, cb, wab, wba, wsa, wsb, ba, bb,
                a_out, b_out):
    agg_a = sa[...] / jnp.maximum(ca[...], 1.0)
    agg_b = sb[...] / jnp.maximum(cb[...], 1.0)
    f32 = jnp.float32
    na = (jnp.dot(xa[...], wsa[...], preferred_element_type=f32)
          + jnp.dot(agg_a, wba[...], preferred_element_type=f32) + ba[...])
    nb = (jnp.dot(xb[...], wsb[...], preferred_element_type=f32)
          + jnp.dot(agg_b, wab[...], preferred_element_type=f32) + bb[...])
    a_out[...] = jnp.maximum(na, 0.0)
    b_out[...] = jnp.maximum(nb, 0.0)


_BLK = 1000
_GRID = _N // _BLK


def _node_specs():
    rows = pl.BlockSpec((_BLK, _D), lambda i: (i, 0))
    cnt = pl.BlockSpec((_BLK, 1), lambda i: (i, 0))
    full = pl.BlockSpec((_D, _D), lambda i: (0, 0))
    bias = pl.BlockSpec((1, _D), lambda i: (0, 0))
    return rows, cnt, full, bias


def _tc_layer(xa, xb, sa, sb, ca, cb, wab, wba, wsa, wsb, ba, bb):
    rows, cnt, full, bias = _node_specs()
    return pl.pallas_call(
        _layer_body,
        grid=(_GRID,),
        in_specs=[rows, rows, rows, rows, cnt, cnt,
                  full, full, full, full, bias, bias],
        out_specs=[rows, rows],
        out_shape=[jax.ShapeDtypeStruct((_N, _D), jnp.float32)] * 2,
    )(xa, xb, sa, sb, ca, cb, wab, wba, wsa, wsb, ba, bb)


def _layer2_body(xa, xb, sa, sb, ca, cb, wab, wba, wsa, wsb, ba, bb,
                 wh, bh, wo, bo, out, pa, pb):
    i = pl.program_id(0)
    agg_a = sa[...] / jnp.maximum(ca[...], 1.0)
    agg_b = sb[...] / jnp.maximum(cb[...], 1.0)
    f32 = jnp.float32
    na = (jnp.dot(xa[...], wsa[...], preferred_element_type=f32)
          + jnp.dot(agg_a, wba[...], preferred_element_type=f32) + ba[...])
    nb = (jnp.dot(xb[...], wsb[...], preferred_element_type=f32)
          + jnp.dot(agg_b, wab[...], preferred_element_type=f32) + bb[...])
    na = jnp.maximum(na, 0.0)
    nb = jnp.maximum(nb, 0.0)

    @pl.when(i == 0)
    def _():
        pa[...] = jnp.zeros_like(pa)
        pb[...] = jnp.zeros_like(pb)

    pa[...] += jnp.sum(na, axis=0, keepdims=True)
    pb[...] += jnp.sum(nb, axis=0, keepdims=True)

    @pl.when(i == pl.num_programs(0) - 1)
    def _():
        pooled = jnp.concatenate([pa[...], pb[...]], axis=1)
        h = jnp.maximum(
            jnp.dot(pooled, wh[...], preferred_element_type=f32) + bh[...],
            0.0)
        out[...] = jnp.dot(h, wo[...], preferred_element_type=f32) + bo[...]


def _tc_layer2(xa, xb, sa, sb, ca, cb, wab, wba, wsa, wsb, ba, bb,
               wh, bh, wo, bo):
    rows, cnt, full, bias = _node_specs()
    return pl.pallas_call(
        _layer2_body,
        grid=(_GRID,),
        in_specs=[rows, rows, rows, rows, cnt, cnt,
                  full, full, full, full, bias, bias,
                  pl.BlockSpec((2 * _D, _D), lambda i: (0, 0)),
                  bias,
                  pl.BlockSpec((_D, _OUT), lambda i: (0, 0)),
                  pl.BlockSpec((1, _OUT), lambda i: (0, 0))],
        out_specs=pl.BlockSpec((1, _OUT), lambda i: (0, 0)),
        out_shape=jax.ShapeDtypeStruct((1, _OUT), jnp.float32),
        scratch_shapes=[pltpu.VMEM((1, _D), jnp.float32),
                        pltpu.VMEM((1, _D), jnp.float32)],
    )(xa, xb, sa, sb, ca, cb, wab, wba, wsa, wsb, ba, bb, wh, bh, wo, bo)


def _prep_idx(ei):
    src, dst = ei[0], ei[1]
    pad = _EP - _E
    src = jnp.concatenate([src, jnp.zeros((pad,), jnp.int32)])
    dst = jnp.concatenate([dst, jnp.full((pad,), _N, jnp.int32)])
    return jnp.stack([src.reshape(_NS, _NBLK, _CHB, _CH),
                      dst.reshape(_NS, _NBLK, _CHB, _CH)], axis=2)


def kernel(x_a, x_b, edge_index_ab, edge_index_ba, W_ab_0, W_ba_0,
           W_self_a_0, W_self_b_0, b_a_0, b_b_0, W_ab_1, W_ba_1,
           W_self_a_1, W_self_b_1, b_a_1, b_b_1, W_h, b_h, W_o, b_o):
    edge_idx = jnp.stack([_prep_idx(edge_index_ab),
                          _prep_idx(edge_index_ba)])

    z128 = jnp.zeros((_CH, _D), jnp.float32)

    seg0 = _segsum_kernel(True)
    seg1 = _segsum_kernel(False)

    s0, craw = seg0(x_a, x_b, edge_idx, z128)
    cflat = _tc_cnt_reduce(craw)
    c_b = cflat[0, :_N].reshape(_N, 1)
    c_a = cflat[1, :_N].reshape(_N, 1)
    s_b0 = s0[0, :_N]
    s_a0 = s0[1, :_N]

    ba0 = b_a_0.reshape(1, _D)
    bb0 = b_b_0.reshape(1, _D)
    a1, b1 = _tc_layer(x_a, x_b, s_a0, s_b0, c_a, c_b,
                       W_ab_0, W_ba_0, W_self_a_0, W_self_b_0, ba0, bb0)

    (s1,) = seg1(a1, b1, edge_idx, z128)
    s_b1 = s1[0, :_N]
    s_a1 = s1[1, :_N]

    out = _tc_layer2(a1, b1, s_a1, s_b1, c_a, c_b,
                     W_ab_1, W_ba_1, W_self_a_1, W_self_b_1,
                     b_a_1.reshape(1, _D), b_b_1.reshape(1, _D),
                     W_h, b_h.reshape(1, _D), W_o, b_o.reshape(1, _OUT))
    return out.reshape(_OUT)
